# baseline (speedup 1.0000x reference)
import functools

import jax
import jax.numpy as jnp
from jax import lax
from jax.experimental import pallas as pl
from jax.experimental.pallas import tpu as pltpu
from jax.experimental.pallas import tpu_sc as plsc

N = 10000
D = 128
DH = 64
NC = 2
NS = 16
NR = 10112
RPT = NR // NS
SS = RPT // 4
EBLK = 128
N_MACRO = 4


def _mesh():
    return plsc.VectorSubcoreMesh(core_axis_name="c", subcore_axis_name="s")


def _fast_rsqrt(d):
    i = plsc.bitcast(d, jnp.int32)
    y = plsc.bitcast(jnp.int32(0x5F3759DF) - lax.shift_right_logical(i, 1),
                     jnp.float32)
    for _ in range(3):
        y = y * (1.5 - 0.5 * d * y * y)
    return y


def _hist_body(n_blocks, dst_hbm, ones_hbm, zeros_hbm, out_hbm,
               hist_sp, ones_v, didx, asem):
    c = lax.axis_index("c")
    s = lax.axis_index("s")
    pltpu.sync_copy(zeros_hbm, hist_sp.at[pl.ds(s * RPT, RPT)])
    pltpu.sync_copy(ones_hbm, ones_v)
    wid = c * NS + s
    pltpu.sync_copy(dst_hbm.at[pl.ds(wid * n_blocks, n_blocks)], didx)
    plsc.subcore_barrier()

    win = 8

    def blk(i, carry):
        pltpu.async_copy(ones_v, hist_sp.at[didx.at[i]], asem, add=True)

        @pl.when(i >= win)
        def _():
            pltpu.make_async_copy(ones_v, hist_sp.at[didx.at[i]], asem).wait()
        return carry

    lax.fori_loop(0, n_blocks, blk, 0)

    def drain(i, carry):
        pltpu.make_async_copy(ones_v, hist_sp.at[didx.at[0]], asem).wait()
        return carry

    lax.fori_loop(0, min(win, n_blocks), drain, 0)
    plsc.subcore_barrier()
    pltpu.sync_copy(hist_sp.at[pl.ds(s * RPT, RPT)],
                    out_hbm.at[c, pl.ds(s * RPT, RPT)])


def _hist(dst2d, ones, zeros):
    n_blocks = dst2d.shape[0] // (NC * NS)
    body = functools.partial(_hist_body, n_blocks)
    return pl.kernel(
        body,
        out_type=jax.ShapeDtypeStruct((NC, NR, 16), jnp.float32),
        mesh=_mesh(),
        compiler_params=pltpu.CompilerParams(use_tc_tiling_on_sc=False),
        scratch_types=[
            pltpu.VMEM_SHARED((NR, 16), jnp.float32),
            pltpu.VMEM((EBLK, 16), jnp.float32),
            pltpu.VMEM((n_blocks, EBLK), jnp.int32),
            pltpu.SemaphoreType.DMA,
        ],
    )(dst2d, ones, zeros)


def _row_scale(fbuf, hbuf, kind):
    def row(n, carry):
        deg = jnp.maximum(hbuf[0, n] + hbuf[1, n], 1.0)
        sc = _fast_rsqrt(deg) if kind == "norm" else 1.0 / deg
        for q in range(DH // 16):
            fbuf[n, pl.ds(q * 16, 16)] = fbuf[n, pl.ds(q * 16, 16)] * sc
        return carry

    lax.fori_loop(0, SS, row, 0)


def _hop_body(n_blocks, mode_in, mode_out, x_hbm, hist_hbm, src_hbm, dst_hbm,
              zeros_hbm, out_hbm, g_sp, acc_sp, hbuf, fbuf, sidx, didx, rows,
              gsem, ssem):
    c = lax.axis_index("c")
    s = lax.axis_index("s")
    r0 = s * RPT

    pltpu.sync_copy(zeros_hbm, acc_sp.at[pl.ds(r0, RPT)])
    for k in range(RPT // SS):
        rk = r0 + k * SS
        if mode_in == "norm":
            pltpu.sync_copy(x_hbm.at[pl.ds(rk, SS), pl.ds(c * DH, DH)], fbuf)
        else:
            pltpu.sync_copy(x_hbm.at[c, pl.ds(rk, SS)], fbuf)
        pltpu.sync_copy(hist_hbm.at[0, pl.ds(rk, SS)], hbuf.at[0])
        pltpu.sync_copy(hist_hbm.at[1, pl.ds(rk, SS)], hbuf.at[1])
        _row_scale(fbuf, hbuf, "norm" if mode_in == "norm" else "inv")
        pltpu.sync_copy(fbuf, g_sp.at[pl.ds(rk, SS)])
    plsc.subcore_barrier()

    mchunk = n_blocks // N_MACRO

    def macro(m, mcarry):
        b0 = s * n_blocks + m * mchunk
        pltpu.sync_copy(src_hbm.at[pl.ds(b0, mchunk)], sidx)
        pltpu.sync_copy(dst_hbm.at[pl.ds(b0, mchunk)], didx)
        pltpu.async_copy(g_sp.at[sidx.at[0]], rows.at[0], gsem.at[0])

        def blk(i, carry):
            j = lax.rem(i, 2)
            jn = lax.rem(i + 1, 2)

            @pl.when(i + 1 < mchunk)
            def _():
                @pl.when(i >= 1)
                def _():
                    pltpu.make_async_copy(
                        rows.at[jn], acc_sp.at[didx.at[i]],
                        ssem.at[jn]).wait()
                pltpu.async_copy(g_sp.at[sidx.at[i + 1]], rows.at[jn],
                                 gsem.at[jn])

            pltpu.make_async_copy(g_sp.at[sidx.at[i]], rows.at[j],
                                  gsem.at[j]).wait()
            pltpu.async_copy(rows.at[j], acc_sp.at[didx.at[i]], ssem.at[j],
                             add=True)
            return carry

        lax.fori_loop(0, mchunk, blk, 0)
        j_last = (mchunk - 1) % 2
        pltpu.make_async_copy(rows.at[j_last], acc_sp.at[didx.at[0]],
                              ssem.at[j_last]).wait()
        pltpu.make_async_copy(rows.at[1 - j_last], acc_sp.at[didx.at[0]],
                              ssem.at[1 - j_last]).wait()
        return mcarry

    lax.fori_loop(0, N_MACRO, macro, 0)
    plsc.subcore_barrier()

    if mode_out == "none":
        pltpu.sync_copy(acc_sp.at[pl.ds(r0, RPT)],
                        out_hbm.at[c, pl.ds(r0, RPT)])
    else:
        for k in range(RPT // SS):
            rk = r0 + k * SS
            pltpu.sync_copy(acc_sp.at[pl.ds(rk, SS)], fbuf)
            pltpu.sync_copy(hist_hbm.at[0, pl.ds(rk, SS)], hbuf.at[0])
            pltpu.sync_copy(hist_hbm.at[1, pl.ds(rk, SS)], hbuf.at[1])
            _row_scale(fbuf, hbuf, "norm")
            pltpu.sync_copy(fbuf, out_hbm.at[c, pl.ds(rk, SS)])


def _hop(x, hist, src2d, dst2d, zeros, mode_in, mode_out):
    n_blocks = src2d.shape[0] // NS
    assert n_blocks % N_MACRO == 0
    mchunk = n_blocks // N_MACRO
    body = functools.partial(_hop_body, n_blocks, mode_in, mode_out)
    return pl.kernel(
        body,
        out_type=jax.ShapeDtypeStruct((NC, NR, DH), jnp.float32),
        mesh=_mesh(),
        compiler_params=pltpu.CompilerParams(
            use_tc_tiling_on_sc=False, needs_layout_passes=False),
        scratch_types=[
            pltpu.VMEM_SHARED((NR, DH), jnp.float32),
            pltpu.VMEM_SHARED((NR, DH), jnp.float32),
            pltpu.VMEM((2, SS, 16), jnp.float32),
            pltpu.VMEM((SS, DH), jnp.float32),
            pltpu.VMEM((mchunk, EBLK), jnp.int32),
            pltpu.VMEM((mchunk, EBLK), jnp.int32),
            pltpu.VMEM((2, EBLK, DH), jnp.float32),
            pltpu.SemaphoreType.DMA((2,)),
            pltpu.SemaphoreType.DMA((2,)),
        ],
    )(x, hist, src2d, dst2d, zeros)


def _final_body(y_ref, w_ref, b_ref, o_ref):
    h = jnp.concatenate([y_ref[0, :N], y_ref[1, :N]], axis=1)
    mean = jnp.mean(h, axis=0)
    cen = h - mean[None, :]
    var = jnp.sum(cen * cen, axis=0) / (N - 1)
    xn = cen / jnp.sqrt(var)[None, :]
    out = lax.dot_general(xn, w_ref[...], (((1,), (1,)), ((), ())),
                          preferred_element_type=jnp.float32)
    o_ref[...] = out + b_ref[...][None, :]


def _final(y, W, b):
    return pl.pallas_call(
        _final_body,
        out_shape=jax.ShapeDtypeStruct((N, D), jnp.float32),
    )(y, W, b)


def kernel(feat, edge_index, W, b):
    E = edge_index.shape[1]
    quant = NS * EBLK * N_MACRO
    e_pad = ((E + quant - 1) // quant) * quant
    pad = jnp.full((e_pad - E,), N, dtype=jnp.int32)
    src = jnp.concatenate([edge_index[0].astype(jnp.int32), pad]).reshape(-1, EBLK)
    dst = jnp.concatenate([edge_index[1].astype(jnp.int32), pad]).reshape(-1, EBLK)

    feat_pad = jnp.concatenate(
        [feat, jnp.zeros((NR - N, D), jnp.float32)], axis=0)
    ones = jnp.ones((EBLK, 16), jnp.float32)
    zeros_h = jnp.zeros((RPT, 16), jnp.float32)
    zeros_c = jnp.zeros((RPT, DH), jnp.float32)

    hist = _hist(dst, ones, zeros_h)
    y1 = _hop(feat_pad, hist, src, dst, zeros_c, "norm", "none")
    y2 = _hop(y1, hist, src, dst, zeros_c, "inv", "norm")
    return _final(y2, W, b)


# speedup vs baseline: 1.0809x; 1.0809x over previous
import functools

import jax
import jax.numpy as jnp
from jax import lax
from jax.experimental import pallas as pl
from jax.experimental.pallas import tpu as pltpu
from jax.experimental.pallas import tpu_sc as plsc

N = 10000
D = 128
DH = 64
NC = 2
NS = 16
NR = 10112
RPT = NR // NS
SS = RPT // 8
EBLK = 128
N_MACRO = 4


def _mesh():
    return plsc.VectorSubcoreMesh(core_axis_name="c", subcore_axis_name="s")


def _fast_rsqrt(d):
    i = plsc.bitcast(d, jnp.int32)
    y = plsc.bitcast(jnp.int32(0x5F3759DF) - lax.shift_right_logical(i, 1),
                     jnp.float32)
    for _ in range(2):
        y = y * (1.5 - 0.5 * d * y * y)
    return y


def _hist_body(n_blocks, dst_hbm, ones_hbm, zeros_hbm, out_hbm,
               hist_sp, ones_v, didx, asem):
    c = lax.axis_index("c")
    s = lax.axis_index("s")
    pltpu.sync_copy(zeros_hbm, hist_sp.at[pl.ds(s * RPT, RPT)])
    pltpu.sync_copy(ones_hbm, ones_v)
    wid = c * NS + s
    pltpu.sync_copy(dst_hbm.at[pl.ds(wid * n_blocks, n_blocks)], didx)
    plsc.subcore_barrier()

    win = 8

    def blk(i, carry):
        pltpu.async_copy(ones_v, hist_sp.at[didx.at[i]], asem, add=True)

        @pl.when(i >= win)
        def _():
            pltpu.make_async_copy(ones_v, hist_sp.at[didx.at[i]], asem).wait()
        return carry

    lax.fori_loop(0, n_blocks, blk, 0)

    def drain(i, carry):
        pltpu.make_async_copy(ones_v, hist_sp.at[didx.at[0]], asem).wait()
        return carry

    lax.fori_loop(0, min(win, n_blocks), drain, 0)
    plsc.subcore_barrier()
    pltpu.sync_copy(hist_sp.at[pl.ds(s * RPT, RPT)],
                    out_hbm.at[c, pl.ds(s * RPT, RPT)])


def _hist(dst2d, ones, zeros):
    n_blocks = dst2d.shape[0] // (NC * NS)
    body = functools.partial(_hist_body, n_blocks)
    return pl.kernel(
        body,
        out_type=jax.ShapeDtypeStruct((NC, NR, 16), jnp.float32),
        mesh=_mesh(),
        compiler_params=pltpu.CompilerParams(use_tc_tiling_on_sc=False),
        scratch_types=[
            pltpu.VMEM_SHARED((NR, 16), jnp.float32),
            pltpu.VMEM((EBLK, 16), jnp.float32),
            pltpu.VMEM((n_blocks, EBLK), jnp.int32),
            pltpu.SemaphoreType.DMA,
        ],
    )(dst2d, ones, zeros)


def _scale_rows(fbuf, norm_v, base, power):
    @plsc.parallel_loop(0, SS, 1, unroll=8)
    def row(n):
        sc = norm_v[base + n]
        if power == 2:
            sc = sc * sc
        for q in range(DH // 16):
            fbuf[n, pl.ds(q * 16, 16)] = fbuf[n, pl.ds(q * 16, 16)] * sc


def _bighop_body(n_blocks, feat_hbm, hist_hbm, src_hbm, dst_hbm, zeros_hbm,
                 out_hbm, g_sp, acc_sp, norm_v, hbuf, fbuf, sidx, didx, rows,
                 gsem, ssem):
    c = lax.axis_index("c")
    s = lax.axis_index("s")
    r0 = s * RPT

    pltpu.sync_copy(zeros_hbm, acc_sp.at[pl.ds(r0, RPT)])
    for k in range(RPT // SS):
        rk = r0 + k * SS
        pltpu.sync_copy(hist_hbm.at[0, pl.ds(rk, SS)], hbuf.at[0])
        pltpu.sync_copy(hist_hbm.at[1, pl.ds(rk, SS)], hbuf.at[1])

        @plsc.parallel_loop(0, SS, 1, unroll=8)
        def row(n):
            norm_v[k * SS + n] = _fast_rsqrt(
                jnp.maximum(hbuf[0, n] + hbuf[1, n], 1.0))

        pltpu.sync_copy(feat_hbm.at[pl.ds(rk, SS), pl.ds(c * DH, DH)], fbuf)
        _scale_rows(fbuf, norm_v, k * SS, 1)
        pltpu.sync_copy(fbuf, g_sp.at[pl.ds(rk, SS)])
    plsc.subcore_barrier()

    mchunk = n_blocks // N_MACRO

    def hop():
        def macro(m, mcarry):
            b0 = s * n_blocks + m * mchunk
            pltpu.sync_copy(src_hbm.at[pl.ds(b0, mchunk)], sidx)
            pltpu.sync_copy(dst_hbm.at[pl.ds(b0, mchunk)], didx)
            pltpu.async_copy(g_sp.at[sidx.at[0]], rows.at[0], gsem.at[0])

            def blk(i, carry):
                j = lax.rem(i, 2)
                jn = lax.rem(i + 1, 2)

                @pl.when(i + 1 < mchunk)
                def _():
                    @pl.when(i >= 1)
                    def _():
                        pltpu.make_async_copy(
                            rows.at[jn], acc_sp.at[didx.at[i]],
                            ssem.at[jn]).wait()
                    pltpu.async_copy(g_sp.at[sidx.at[i + 1]], rows.at[jn],
                                     gsem.at[jn])

                pltpu.make_async_copy(g_sp.at[sidx.at[i]], rows.at[j],
                                      gsem.at[j]).wait()
                pltpu.async_copy(rows.at[j], acc_sp.at[didx.at[i]],
                                 ssem.at[j], add=True)
                return carry

            lax.fori_loop(0, mchunk, blk, 0)
            j_last = (mchunk - 1) % 2
            pltpu.make_async_copy(rows.at[j_last], acc_sp.at[didx.at[0]],
                                  ssem.at[j_last]).wait()
            pltpu.make_async_copy(rows.at[1 - j_last], acc_sp.at[didx.at[0]],
                                  ssem.at[1 - j_last]).wait()
            return mcarry

        lax.fori_loop(0, N_MACRO, macro, 0)

    hop()
    plsc.subcore_barrier()

    for k in range(RPT // SS):
        rk = r0 + k * SS
        pltpu.sync_copy(acc_sp.at[pl.ds(rk, SS)], fbuf)
        _scale_rows(fbuf, norm_v, k * SS, 2)
        pltpu.sync_copy(fbuf, g_sp.at[pl.ds(rk, SS)])
    pltpu.sync_copy(zeros_hbm, acc_sp.at[pl.ds(r0, RPT)])
    plsc.subcore_barrier()

    hop()
    plsc.subcore_barrier()

    for k in range(RPT // SS):
        rk = r0 + k * SS
        pltpu.sync_copy(acc_sp.at[pl.ds(rk, SS)], fbuf)
        _scale_rows(fbuf, norm_v, k * SS, 1)
        pltpu.sync_copy(fbuf, out_hbm.at[c, pl.ds(rk, SS)])


def _bighop(feat_pad, hist, src2d, dst2d, zeros):
    n_blocks = src2d.shape[0] // NS
    assert n_blocks % N_MACRO == 0
    mchunk = n_blocks // N_MACRO
    body = functools.partial(_bighop_body, n_blocks)
    return pl.kernel(
        body,
        out_type=jax.ShapeDtypeStruct((NC, NR, DH), jnp.float32),
        mesh=_mesh(),
        compiler_params=pltpu.CompilerParams(
            use_tc_tiling_on_sc=False, needs_layout_passes=False),
        scratch_types=[
            pltpu.VMEM_SHARED((NR, DH), jnp.float32),
            pltpu.VMEM_SHARED((NR, DH), jnp.float32),
            pltpu.VMEM((RPT, 16), jnp.float32),
            pltpu.VMEM((2, SS, 16), jnp.float32),
            pltpu.VMEM((SS, DH), jnp.float32),
            pltpu.VMEM((mchunk, EBLK), jnp.int32),
            pltpu.VMEM((mchunk, EBLK), jnp.int32),
            pltpu.VMEM((2, EBLK, DH), jnp.float32),
            pltpu.SemaphoreType.DMA((2,)),
            pltpu.SemaphoreType.DMA((2,)),
        ],
    )(feat_pad, hist, src2d, dst2d, zeros)


def _final_body(y_ref, w_ref, b_ref, o_ref):
    h = jnp.concatenate([y_ref[0, :N], y_ref[1, :N]], axis=1)
    mean = jnp.mean(h, axis=0)
    cen = h - mean[None, :]
    var = jnp.sum(cen * cen, axis=0) / (N - 1)
    xn = cen / jnp.sqrt(var)[None, :]
    out = lax.dot_general(xn, w_ref[...], (((1,), (1,)), ((), ())),
                          preferred_element_type=jnp.float32)
    o_ref[...] = out + b_ref[...][None, :]


def _final(y, W, b):
    return pl.pallas_call(
        _final_body,
        out_shape=jax.ShapeDtypeStruct((N, D), jnp.float32),
    )(y, W, b)


def kernel(feat, edge_index, W, b):
    E = edge_index.shape[1]
    quant = NS * EBLK * N_MACRO
    e_pad = ((E + quant - 1) // quant) * quant
    pad = jnp.full((e_pad - E,), N, dtype=jnp.int32)
    src = jnp.concatenate([edge_index[0].astype(jnp.int32), pad]).reshape(-1, EBLK)
    dst = jnp.concatenate([edge_index[1].astype(jnp.int32), pad]).reshape(-1, EBLK)

    feat_pad = jnp.concatenate(
        [feat, jnp.zeros((NR - N, D), jnp.float32)], axis=0)
    ones = jnp.ones((EBLK, 16), jnp.float32)
    zeros_h = jnp.zeros((RPT, 16), jnp.float32)
    zeros_c = jnp.zeros((RPT, DH), jnp.float32)

    hist = _hist(dst, ones, zeros_h)
    y2 = _bighop(feat_pad, hist, src, dst, zeros_c)
    return _final(y2, W, b)
